# Initial kernel scaffold; baseline (speedup 1.0000x reference)
import functools

import jax
import jax.numpy as jnp
from jax import lax
from jax.experimental import pallas as pl
from jax.experimental.pallas import tpu as pltpu
from jax.experimental.pallas import tpu_sc as plsc

NC = 2
NS = 16
NW = NC * NS
CH = 125


@functools.cache
def _make_agg(n, d, e):
    chunks_total = e // CH
    chunks_per_tile = chunks_total // NW
    rows_per_tile = n // NS
    reps = rows_per_tile // CH

    mesh = plsc.VectorSubcoreMesh(
        core_axis_name="c", subcore_axis_name="s", num_cores=NC, num_subcores=NS
    )

    @functools.partial(
        pl.kernel,
        out_type=jax.ShapeDtypeStruct((NC, n, d), jnp.float32),
        mesh=mesh,
        scratch_types=[
            pltpu.VMEM((chunks_per_tile, CH), jnp.int32),
            pltpu.VMEM((chunks_per_tile, CH), jnp.int32),
            pltpu.VMEM((CH, d), jnp.float32),
            pltpu.VMEM_SHARED((n, d), jnp.float32),
            pltpu.SemaphoreType.DMA,
        ],
    )
    def agg(x_hbm, src_hbm, dst_hbm, out_hbm, src_v, dst_v, rows_v, acc_sh, sem):
        c = lax.axis_index("c")
        s = lax.axis_index("s")
        wid = c * NS + s

        pltpu.sync_copy(
            src_hbm.at[pl.ds(wid * chunks_per_tile, chunks_per_tile)], src_v
        )
        pltpu.sync_copy(
            dst_hbm.at[pl.ds(wid * chunks_per_tile, chunks_per_tile)], dst_v
        )

        def zrow(i, carry):
            for jj in range(d // 16):
                rows_v[i, pl.ds(jj * 16, 16)] = jnp.zeros((16,), jnp.float32)
            return carry

        lax.fori_loop(0, CH, zrow, 0)
        for r in range(reps):
            pltpu.sync_copy(
                rows_v, acc_sh.at[pl.ds(s * rows_per_tile + r * CH, CH)]
            )
        plsc.subcore_barrier()

        def body(j, carry):
            pltpu.async_copy(x_hbm.at[src_v.at[j]], rows_v, sem).wait()
            pltpu.sync_copy(rows_v, acc_sh.at[dst_v.at[j]], add=True)
            return carry

        lax.fori_loop(0, chunks_per_tile, body, 0)
        plsc.subcore_barrier()

        pltpu.sync_copy(
            acc_sh.at[pl.ds(s * rows_per_tile, rows_per_tile)],
            out_hbm.at[c].at[pl.ds(s * rows_per_tile, rows_per_tile)],
        )

    return agg


def _gin_dense_body(n, x_ref, a_ref, w1_ref, b1_ref, g_ref, be_ref, w2_ref,
                    b2_ref, out_ref):
    sfull = x_ref[...] + a_ref[0] + a_ref[1]
    t = jnp.dot(sfull, w1_ref[...], preferred_element_type=jnp.float32)
    t = t + b1_ref[...]
    mu = jnp.sum(t, axis=0, keepdims=True) * (1.0 / n)
    ctr = t - mu
    var = jnp.sum(ctr * ctr, axis=0, keepdims=True) * (1.0 / n)
    h = ctr * lax.rsqrt(var + 1e-5) * g_ref[...] + be_ref[...]
    h = jnp.maximum(h, 0.0)
    h = jnp.dot(h, w2_ref[...], preferred_element_type=jnp.float32) + b2_ref[...]
    out_ref[...] = jnp.maximum(h, 0.0)


@functools.cache
def _make_gin_dense(n, din, h):
    return pl.pallas_call(
        functools.partial(_gin_dense_body, n),
        out_shape=jax.ShapeDtypeStruct((n, h), jnp.float32),
    )


def _pool_head_body(n, g, h1_ref, h2_ref, h3_ref, batch_ref, fc1w_ref,
                    fc1b_ref, fc2w_ref, fc2b_ref, out_ref):
    b = batch_ref[...]
    gids = lax.broadcasted_iota(jnp.int32, (g, n), 0)
    sel = jnp.where(gids == b, 1.0, 0.0)
    p1 = jnp.dot(sel, h1_ref[...], preferred_element_type=jnp.float32)
    p2 = jnp.dot(sel, h2_ref[...], preferred_element_type=jnp.float32)
    p3 = jnp.dot(sel, h3_ref[...], preferred_element_type=jnp.float32)
    cat = jnp.concatenate([p1, p2, p3], axis=1)
    y = jnp.dot(cat, fc1w_ref[...], preferred_element_type=jnp.float32)
    y = jnp.maximum(y + fc1b_ref[...], 0.0)
    y = jnp.dot(y, fc2w_ref[...], preferred_element_type=jnp.float32)
    y = y + fc2b_ref[...]
    m = jnp.max(y, axis=1, keepdims=True)
    ex = jnp.exp(y - m)
    out_ref[...] = (y - m) - jnp.log(jnp.sum(ex, axis=1, keepdims=True))


@functools.cache
def _make_pool_head(n, g, h, out):
    return pl.pallas_call(
        functools.partial(_pool_head_body, n, g),
        out_shape=jax.ShapeDtypeStruct((g, out), jnp.float32),
    )


def kernel(x, edge_index, batch, c1_W1, c1_b1, c1_g, c1_be, c1_W2, c1_b2,
           c2_W1, c2_b1, c2_g, c2_be, c2_W2, c2_b2, c3_W1, c3_b1, c3_g,
           c3_be, c3_W2, c3_b2, fc1_W, fc1_b, fc2_W, fc2_b):
    n, d = x.shape
    e = edge_index.shape[1]
    h = c1_W1.shape[1]
    g = 64
    out = fc2_W.shape[1]

    src = edge_index[0].reshape(e // CH, CH)
    dst = edge_index[1].reshape(e // CH, CH)

    agg_d = _make_agg(n, d, e)
    agg_h = _make_agg(n, h, e)
    dense1 = _make_gin_dense(n, d, h)
    dense23 = _make_gin_dense(n, h, h)
    pool_head = _make_pool_head(n, g, h, out)

    def layer(maker, dense, xin, w1, b1, gm, be, w2, b2):
        parts = maker(xin, src, dst)
        return dense(xin, parts, w1, b1.reshape(1, h), gm.reshape(1, h),
                     be.reshape(1, h), w2, b2.reshape(1, h))

    h1 = layer(agg_d, dense1, x, c1_W1, c1_b1, c1_g, c1_be, c1_W2, c1_b2)
    h2 = layer(agg_h, dense23, h1, c2_W1, c2_b1, c2_g, c2_be, c2_W2, c2_b2)
    h3 = layer(agg_h, dense23, h2, c3_W1, c3_b1, c3_g, c3_be, c3_W2, c3_b2)

    return pool_head(h1, h2, h3, batch.reshape(1, n), fc1_W,
                     fc1_b.reshape(1, 3 * h), fc2_W, fc2_b.reshape(1, out))


# speedup vs baseline: 7.5665x; 7.5665x over previous
import functools

import jax
import jax.numpy as jnp
from jax import lax
from jax.experimental import pallas as pl
from jax.experimental.pallas import tpu as pltpu
from jax.experimental.pallas import tpu_sc as plsc

NC = 2
NS = 16
NW = NC * NS
CH = 125


ZB = 128


@functools.cache
def _make_agg(n_src, n_dst, d, e):
    chunks_total = e // CH
    chunks_per_tile = chunks_total // NW
    rows_per_tile = -(-(-(-n_dst // NS)) // ZB) * ZB
    n_pad = rows_per_tile * NS
    reps = rows_per_tile // ZB

    mesh = plsc.VectorSubcoreMesh(
        core_axis_name="c", subcore_axis_name="s", num_cores=NC, num_subcores=NS
    )

    @functools.partial(
        pl.kernel,
        out_type=jax.ShapeDtypeStruct((NC, n_pad, d), jnp.float32),
        mesh=mesh,
        scratch_types=[
            pltpu.VMEM((chunks_per_tile, CH), jnp.int32),
            pltpu.VMEM((chunks_per_tile, CH), jnp.int32),
            pltpu.VMEM((CH, d), jnp.float32),
            pltpu.VMEM((ZB, d), jnp.float32),
            pltpu.VMEM_SHARED((n_pad, d), jnp.float32),
            pltpu.SemaphoreType.DMA,
        ],
        compiler_params=pltpu.CompilerParams(use_tc_tiling_on_sc=False),
    )
    def agg(x_hbm, src_hbm, dst_hbm, out_hbm, src_v, dst_v, rows_v, zb_v,
            acc_sh, sem):
        c = lax.axis_index("c")
        s = lax.axis_index("s")
        wid = c * NS + s

        pltpu.sync_copy(
            src_hbm.at[pl.ds(wid * chunks_per_tile, chunks_per_tile)], src_v
        )
        pltpu.sync_copy(
            dst_hbm.at[pl.ds(wid * chunks_per_tile, chunks_per_tile)], dst_v
        )

        def zrow(i, carry):
            for jj in range(d // 16):
                zb_v[i, pl.ds(jj * 16, 16)] = jnp.zeros((16,), jnp.float32)
            return carry

        lax.fori_loop(0, ZB, zrow, 0)
        for r in range(reps):
            pltpu.sync_copy(
                zb_v, acc_sh.at[pl.ds(s * rows_per_tile + r * ZB, ZB)]
            )
        plsc.subcore_barrier()

        def body(j, carry):
            pltpu.async_copy(x_hbm.at[src_v.at[j]], rows_v, sem).wait()
            pltpu.sync_copy(rows_v, acc_sh.at[dst_v.at[j]], add=True)
            return carry

        lax.fori_loop(0, chunks_per_tile, body, 0)
        plsc.subcore_barrier()

        pltpu.sync_copy(
            acc_sh.at[pl.ds(s * rows_per_tile, rows_per_tile)],
            out_hbm.at[c].at[pl.ds(s * rows_per_tile, rows_per_tile)],
        )

    return agg


def _gin_dense_body(n, x_ref, a_ref, w1_ref, b1_ref, g_ref, be_ref, w2_ref,
                    b2_ref, out_ref):
    sfull = x_ref[...] + a_ref[0][:n] + a_ref[1][:n]
    _gin_mlp(n, sfull, w1_ref, b1_ref, g_ref, be_ref, w2_ref, b2_ref, out_ref)


def _gin_dense1_body(n, x_ref, pa_ref, pb_ref, w1_ref, b1_ref, g_ref, be_ref,
                     w2_ref, b2_ref, out_ref):
    agg = jnp.concatenate(
        [pa_ref[0][:n] + pa_ref[1][:n], pb_ref[0][:n] + pb_ref[1][:n]], axis=1
    )
    sfull = x_ref[...] + agg
    _gin_mlp(n, sfull, w1_ref, b1_ref, g_ref, be_ref, w2_ref, b2_ref, out_ref)


def _gin_mlp(n, sfull, w1_ref, b1_ref, g_ref, be_ref, w2_ref, b2_ref, out_ref):
    t = jnp.dot(sfull, w1_ref[...], preferred_element_type=jnp.float32)
    t = t + b1_ref[...]
    mu = jnp.sum(t, axis=0, keepdims=True) * (1.0 / n)
    ctr = t - mu
    var = jnp.sum(ctr * ctr, axis=0, keepdims=True) * (1.0 / n)
    h = ctr * lax.rsqrt(var + 1e-5) * g_ref[...] + be_ref[...]
    h = jnp.maximum(h, 0.0)
    h = jnp.dot(h, w2_ref[...], preferred_element_type=jnp.float32) + b2_ref[...]
    out_ref[...] = jnp.maximum(h, 0.0)


@functools.cache
def _make_gin_dense(n, h):
    return pl.pallas_call(
        functools.partial(_gin_dense_body, n),
        out_shape=jax.ShapeDtypeStruct((n, h), jnp.float32),
    )


@functools.cache
def _make_gin_dense1(n, h):
    return pl.pallas_call(
        functools.partial(_gin_dense1_body, n),
        out_shape=jax.ShapeDtypeStruct((n, h), jnp.float32),
    )


def _pool_head_body(n, g, h1_ref, h2_ref, h3_ref, batch_ref, fc1w_ref,
                    fc1b_ref, fc2w_ref, fc2b_ref, out_ref):
    b = batch_ref[...]
    gids = lax.broadcasted_iota(jnp.int32, (g, n), 0)
    sel = jnp.where(gids == b, 1.0, 0.0)
    p1 = jnp.dot(sel, h1_ref[...], preferred_element_type=jnp.float32)
    p2 = jnp.dot(sel, h2_ref[...], preferred_element_type=jnp.float32)
    p3 = jnp.dot(sel, h3_ref[...], preferred_element_type=jnp.float32)
    cat = jnp.concatenate([p1, p2, p3], axis=1)
    y = jnp.dot(cat, fc1w_ref[...], preferred_element_type=jnp.float32)
    y = jnp.maximum(y + fc1b_ref[...], 0.0)
    y = jnp.dot(y, fc2w_ref[...], preferred_element_type=jnp.float32)
    y = y + fc2b_ref[...]
    m = jnp.max(y, axis=1, keepdims=True)
    ex = jnp.exp(y - m)
    out_ref[...] = (y - m) - jnp.log(jnp.sum(ex, axis=1, keepdims=True))


@functools.cache
def _make_pool_head(n, g, h, out):
    return pl.pallas_call(
        functools.partial(_pool_head_body, n, g),
        out_shape=jax.ShapeDtypeStruct((g, out), jnp.float32),
    )


def kernel(x, edge_index, batch, c1_W1, c1_b1, c1_g, c1_be, c1_W2, c1_b2,
           c2_W1, c2_b1, c2_g, c2_be, c2_W2, c2_b2, c3_W1, c3_b1, c3_g,
           c3_be, c3_W2, c3_b2, fc1_W, fc1_b, fc2_W, fc2_b):
    n, d = x.shape
    e = edge_index.shape[1]
    h = c1_W1.shape[1]
    g = 64
    out = fc2_W.shape[1]
    nb = d // h

    src = edge_index[0]
    dst = edge_index[1].reshape(e // CH, CH)

    x2 = x.reshape(nb * n, h)
    src_blk = [(src * nb + kb).reshape(e // CH, CH) for kb in range(nb)]
    src2d = src.reshape(e // CH, CH)

    agg1 = _make_agg(nb * n, n, h, e)
    agg23 = _make_agg(n, n, h, e)
    dense1 = _make_gin_dense1(n, h)
    dense23 = _make_gin_dense(n, h)
    pool_head = _make_pool_head(n, g, h, out)

    pa = agg1(x2, src_blk[0], dst)
    pb = agg1(x2, src_blk[1], dst)
    h1 = dense1(x, pa, pb, c1_W1, c1_b1.reshape(1, h), c1_g.reshape(1, h),
                c1_be.reshape(1, h), c1_W2, c1_b2.reshape(1, h))

    def layer(xin, w1, b1, gm, be, w2, b2):
        parts = agg23(xin, src2d, dst)
        return dense23(xin, parts, w1, b1.reshape(1, h), gm.reshape(1, h),
                       be.reshape(1, h), w2, b2.reshape(1, h))

    h2 = layer(h1, c2_W1, c2_b1, c2_g, c2_be, c2_W2, c2_b2)
    h3 = layer(h2, c3_W1, c3_b1, c3_g, c3_be, c3_W2, c3_b2)

    return pool_head(h1, h2, h3, batch.reshape(1, n), fc1_W,
                     fc1_b.reshape(1, 3 * h), fc2_W, fc2_b.reshape(1, out))


# speedup vs baseline: 7.7301x; 1.0216x over previous
import functools

import jax
import jax.numpy as jnp
from jax import lax
from jax.experimental import pallas as pl
from jax.experimental.pallas import tpu as pltpu
from jax.experimental.pallas import tpu_sc as plsc

NC = 2
NS = 16
NW = NC * NS
CH = 125
ZB = 128


def _pad_rows(n):
    rows_per_tile = -(-(-(-n // NS)) // ZB) * ZB
    return rows_per_tile, rows_per_tile * NS


@functools.cache
def _make_agg(n_pad, d, e):
    chunks_total = e // CH
    chunks_per_tile = chunks_total // NW
    rows_per_tile = n_pad // NS
    reps = rows_per_tile // ZB

    mesh = plsc.VectorSubcoreMesh(
        core_axis_name="c", subcore_axis_name="s", num_cores=NC, num_subcores=NS
    )

    @functools.partial(
        pl.kernel,
        out_type=jax.ShapeDtypeStruct((NC, n_pad, d), jnp.float32),
        mesh=mesh,
        scratch_types=[
            pltpu.VMEM((chunks_per_tile, CH), jnp.int32),
            pltpu.VMEM((chunks_per_tile, CH), jnp.int32),
            pltpu.VMEM((CH, d), jnp.float32),
            pltpu.VMEM((ZB, d), jnp.float32),
            pltpu.VMEM_SHARED((n_pad, d), jnp.float32),
            pltpu.VMEM_SHARED((n_pad, d), jnp.float32),
            pltpu.SemaphoreType.DMA,
        ],
        compiler_params=pltpu.CompilerParams(use_tc_tiling_on_sc=False),
    )
    def agg(xb_hbm, src_hbm, dst_hbm, out_hbm, src_v, dst_v, rows_v, zb_v,
            stage_sh, acc_sh, sem):
        c = lax.axis_index("c")
        s = lax.axis_index("s")
        wid = c * NS + s
        my_rows = pl.ds(s * rows_per_tile, rows_per_tile)

        pltpu.sync_copy(
            src_hbm.at[pl.ds(wid * chunks_per_tile, chunks_per_tile)], src_v
        )
        pltpu.sync_copy(
            dst_hbm.at[pl.ds(wid * chunks_per_tile, chunks_per_tile)], dst_v
        )
        pltpu.sync_copy(xb_hbm.at[my_rows], stage_sh.at[my_rows])

        def zrow(i, carry):
            for jj in range(d // 16):
                zb_v[i, pl.ds(jj * 16, 16)] = jnp.zeros((16,), jnp.float32)
            return carry

        lax.fori_loop(0, ZB, zrow, 0)
        for r in range(reps):
            pltpu.sync_copy(zb_v, acc_sh.at[pl.ds(s * rows_per_tile + r * ZB, ZB)])
        plsc.subcore_barrier()

        def body(j, carry):
            pltpu.sync_copy(stage_sh.at[src_v.at[j]], rows_v)
            pltpu.sync_copy(rows_v, acc_sh.at[dst_v.at[j]], add=True)
            return carry

        lax.fori_loop(0, chunks_per_tile, body, 0)
        plsc.subcore_barrier()

        pltpu.sync_copy(acc_sh.at[my_rows], out_hbm.at[c].at[my_rows])

    return agg


def _gin_mlp(n, n_pad, sfull, w1_ref, b1_ref, g_ref, be_ref, w2_ref, b2_ref,
             out_ref):
    t = jnp.dot(sfull, w1_ref[...], preferred_element_type=jnp.float32)
    t = t + b1_ref[...]
    mu = jnp.sum(t, axis=0, keepdims=True) * (1.0 / n)
    ctr = t - mu
    var = jnp.sum(ctr * ctr, axis=0, keepdims=True) * (1.0 / n)
    h = ctr * lax.rsqrt(var + 1e-5) * g_ref[...] + be_ref[...]
    h = jnp.maximum(h, 0.0)
    h = jnp.dot(h, w2_ref[...], preferred_element_type=jnp.float32) + b2_ref[...]
    h = jnp.maximum(h, 0.0)
    out_ref[...] = jnp.concatenate(
        [h, jnp.zeros((n_pad - n, h.shape[1]), jnp.float32)], axis=0
    )


def _gin_dense1_body(n, n_pad, x_ref, pa_ref, pb_ref, w1_ref, b1_ref, g_ref,
                     be_ref, w2_ref, b2_ref, out_ref):
    agg = jnp.concatenate(
        [pa_ref[0][:n] + pa_ref[1][:n], pb_ref[0][:n] + pb_ref[1][:n]], axis=1
    )
    sfull = x_ref[...] + agg
    _gin_mlp(n, n_pad, sfull, w1_ref, b1_ref, g_ref, be_ref, w2_ref, b2_ref,
             out_ref)


def _gin_dense_body(n, n_pad, x_ref, a_ref, w1_ref, b1_ref, g_ref, be_ref,
                    w2_ref, b2_ref, out_ref):
    sfull = x_ref[...][:n] + a_ref[0][:n] + a_ref[1][:n]
    _gin_mlp(n, n_pad, sfull, w1_ref, b1_ref, g_ref, be_ref, w2_ref, b2_ref,
             out_ref)


@functools.cache
def _make_gin_dense1(n, n_pad, h):
    return pl.pallas_call(
        functools.partial(_gin_dense1_body, n, n_pad),
        out_shape=jax.ShapeDtypeStruct((n_pad, h), jnp.float32),
    )


@functools.cache
def _make_gin_dense(n, n_pad, h):
    return pl.pallas_call(
        functools.partial(_gin_dense_body, n, n_pad),
        out_shape=jax.ShapeDtypeStruct((n_pad, h), jnp.float32),
    )


def _pool_head_body(n, g, h1_ref, h2_ref, h3_ref, batch_ref, fc1w_ref,
                    fc1b_ref, fc2w_ref, fc2b_ref, out_ref):
    b = batch_ref[...]
    gids = lax.broadcasted_iota(jnp.int32, (g, n), 0)
    sel = jnp.where(gids == b, 1.0, 0.0)
    p1 = jnp.dot(sel, h1_ref[...][:n], preferred_element_type=jnp.float32)
    p2 = jnp.dot(sel, h2_ref[...][:n], preferred_element_type=jnp.float32)
    p3 = jnp.dot(sel, h3_ref[...][:n], preferred_element_type=jnp.float32)
    cat = jnp.concatenate([p1, p2, p3], axis=1)
    y = jnp.dot(cat, fc1w_ref[...], preferred_element_type=jnp.float32)
    y = jnp.maximum(y + fc1b_ref[...], 0.0)
    y = jnp.dot(y, fc2w_ref[...], preferred_element_type=jnp.float32)
    y = y + fc2b_ref[...]
    m = jnp.max(y, axis=1, keepdims=True)
    ex = jnp.exp(y - m)
    out_ref[...] = (y - m) - jnp.log(jnp.sum(ex, axis=1, keepdims=True))


@functools.cache
def _make_pool_head(n, g, h, out):
    return pl.pallas_call(
        functools.partial(_pool_head_body, n, g),
        out_shape=jax.ShapeDtypeStruct((g, out), jnp.float32),
    )


def kernel(x, edge_index, batch, c1_W1, c1_b1, c1_g, c1_be, c1_W2, c1_b2,
           c2_W1, c2_b1, c2_g, c2_be, c2_W2, c2_b2, c3_W1, c3_b1, c3_g,
           c3_be, c3_W2, c3_b2, fc1_W, fc1_b, fc2_W, fc2_b):
    n, d = x.shape
    e = edge_index.shape[1]
    h = c1_W1.shape[1]
    g = 64
    out = fc2_W.shape[1]
    _, n_pad = _pad_rows(n)

    src2d = edge_index[0].reshape(e // CH, CH)
    dst2d = edge_index[1].reshape(e // CH, CH)

    pad = ((0, n_pad - n), (0, 0))
    xa = jnp.pad(x[:, :h], pad)
    xb = jnp.pad(x[:, h:], pad)

    agg = _make_agg(n_pad, h, e)
    dense1 = _make_gin_dense1(n, n_pad, h)
    dense23 = _make_gin_dense(n, n_pad, h)
    pool_head = _make_pool_head(n, g, h, out)

    pa = agg(xa, src2d, dst2d)
    pb = agg(xb, src2d, dst2d)
    h1 = dense1(x, pa, pb, c1_W1, c1_b1.reshape(1, h), c1_g.reshape(1, h),
                c1_be.reshape(1, h), c1_W2, c1_b2.reshape(1, h))

    def layer(xin, w1, b1, gm, be, w2, b2):
        parts = agg(xin, src2d, dst2d)
        return dense23(xin, parts, w1, b1.reshape(1, h), gm.reshape(1, h),
                       be.reshape(1, h), w2, b2.reshape(1, h))

    h2 = layer(h1, c2_W1, c2_b1, c2_g, c2_be, c2_W2, c2_b2)
    h3 = layer(h2, c3_W1, c3_b1, c3_g, c3_be, c3_W2, c3_b2)

    return pool_head(h1, h2, h3, batch.reshape(1, n), fc1_W,
                     fc1_b.reshape(1, 3 * h), fc2_W, fc2_b.reshape(1, out))


# speedup vs baseline: 10.1068x; 1.3075x over previous
import functools

import jax
import jax.numpy as jnp
from jax import lax
from jax.experimental import pallas as pl
from jax.experimental.pallas import tpu as pltpu
from jax.experimental.pallas import tpu_sc as plsc

NC = 2
NS = 16
NW = NC * NS
CH = 125
NBUF = 8
ZB = 64


def _pad_rows(n):
    rows_per_tile = -(-(-(-n // NS)) // ZB) * ZB
    return rows_per_tile, rows_per_tile * NS


@functools.cache
def _make_agg(n_pad, d, e):
    chunks_total = e // CH
    chunks_per_tile = chunks_total // NW
    rows_per_tile = n_pad // NS
    reps = rows_per_tile // ZB

    mesh = plsc.VectorSubcoreMesh(
        core_axis_name="c", subcore_axis_name="s", num_cores=NC, num_subcores=NS
    )

    @functools.partial(
        pl.kernel,
        out_type=jax.ShapeDtypeStruct((NC, n_pad, d), jnp.float32),
        mesh=mesh,
        scratch_types=[
            pltpu.VMEM((chunks_per_tile, CH), jnp.int32),
            pltpu.VMEM((chunks_per_tile, CH), jnp.int32),
            pltpu.VMEM((NBUF, CH, d), jnp.float32),
            pltpu.VMEM((ZB, d), jnp.float32),
            pltpu.VMEM_SHARED((n_pad, d), jnp.float32),
            pltpu.SemaphoreType.DMA,
        ],
        compiler_params=pltpu.CompilerParams(use_tc_tiling_on_sc=False),
    )
    def agg(xb_hbm, src_hbm, dst_hbm, out_hbm, src_v, dst_v, rows_v, zb_v,
            acc_sh, sem):
        c = lax.axis_index("c")
        s = lax.axis_index("s")
        wid = c * NS + s
        my_rows = pl.ds(s * rows_per_tile, rows_per_tile)

        pltpu.sync_copy(
            src_hbm.at[pl.ds(wid * chunks_per_tile, chunks_per_tile)], src_v
        )
        pltpu.sync_copy(
            dst_hbm.at[pl.ds(wid * chunks_per_tile, chunks_per_tile)], dst_v
        )

        def zrow(i, carry):
            for jj in range(d // 16):
                zb_v[i, pl.ds(jj * 16, 16)] = jnp.zeros((16,), jnp.float32)
            return carry

        lax.fori_loop(0, ZB, zrow, 0)
        for r in range(reps):
            pltpu.sync_copy(zb_v, acc_sh.at[pl.ds(s * rows_per_tile + r * ZB, ZB)])
        plsc.subcore_barrier()

        def body(gi, carry):
            gds = [
                pltpu.async_copy(
                    xb_hbm.at[src_v.at[gi * NBUF + b]], rows_v.at[b], sem
                )
                for b in range(NBUF)
            ]
            for gd in gds:
                gd.wait()
            sds = [
                pltpu.async_copy(
                    rows_v.at[b], acc_sh.at[dst_v.at[gi * NBUF + b]], sem,
                    add=True,
                )
                for b in range(NBUF)
            ]
            for sd in sds:
                sd.wait()
            return carry

        lax.fori_loop(0, chunks_per_tile // NBUF, body, 0)
        plsc.subcore_barrier()

        pltpu.sync_copy(acc_sh.at[my_rows], out_hbm.at[c].at[my_rows])

    return agg


def _gin_mlp(n, n_pad, sfull, w1_ref, b1_ref, g_ref, be_ref, w2_ref, b2_ref,
             out_ref):
    t = jnp.dot(sfull, w1_ref[...], preferred_element_type=jnp.float32)
    t = t + b1_ref[...]
    mu = jnp.sum(t, axis=0, keepdims=True) * (1.0 / n)
    ctr = t - mu
    var = jnp.sum(ctr * ctr, axis=0, keepdims=True) * (1.0 / n)
    h = ctr * lax.rsqrt(var + 1e-5) * g_ref[...] + be_ref[...]
    h = jnp.maximum(h, 0.0)
    h = jnp.dot(h, w2_ref[...], preferred_element_type=jnp.float32) + b2_ref[...]
    h = jnp.maximum(h, 0.0)
    out_ref[...] = jnp.concatenate(
        [h, jnp.zeros((n_pad - n, h.shape[1]), jnp.float32)], axis=0
    )


def _gin_dense1_body(n, n_pad, x_ref, pa_ref, pb_ref, w1_ref, b1_ref, g_ref,
                     be_ref, w2_ref, b2_ref, out_ref):
    agg = jnp.concatenate(
        [pa_ref[0][:n] + pa_ref[1][:n], pb_ref[0][:n] + pb_ref[1][:n]], axis=1
    )
    sfull = x_ref[...] + agg
    _gin_mlp(n, n_pad, sfull, w1_ref, b1_ref, g_ref, be_ref, w2_ref, b2_ref,
             out_ref)


def _gin_dense_body(n, n_pad, x_ref, a_ref, w1_ref, b1_ref, g_ref, be_ref,
                    w2_ref, b2_ref, out_ref):
    sfull = x_ref[...][:n] + a_ref[0][:n] + a_ref[1][:n]
    _gin_mlp(n, n_pad, sfull, w1_ref, b1_ref, g_ref, be_ref, w2_ref, b2_ref,
             out_ref)


@functools.cache
def _make_gin_dense1(n, n_pad, h):
    return pl.pallas_call(
        functools.partial(_gin_dense1_body, n, n_pad),
        out_shape=jax.ShapeDtypeStruct((n_pad, h), jnp.float32),
    )


@functools.cache
def _make_gin_dense(n, n_pad, h):
    return pl.pallas_call(
        functools.partial(_gin_dense_body, n, n_pad),
        out_shape=jax.ShapeDtypeStruct((n_pad, h), jnp.float32),
    )


def _pool_head_body(n, g, h1_ref, h2_ref, h3_ref, batch_ref, fc1w_ref,
                    fc1b_ref, fc2w_ref, fc2b_ref, out_ref):
    b = batch_ref[...]
    gids = lax.broadcasted_iota(jnp.int32, (g, n), 0)
    sel = jnp.where(gids == b, 1.0, 0.0)
    p1 = jnp.dot(sel, h1_ref[...][:n], preferred_element_type=jnp.float32)
    p2 = jnp.dot(sel, h2_ref[...][:n], preferred_element_type=jnp.float32)
    p3 = jnp.dot(sel, h3_ref[...][:n], preferred_element_type=jnp.float32)
    cat = jnp.concatenate([p1, p2, p3], axis=1)
    y = jnp.dot(cat, fc1w_ref[...], preferred_element_type=jnp.float32)
    y = jnp.maximum(y + fc1b_ref[...], 0.0)
    y = jnp.dot(y, fc2w_ref[...], preferred_element_type=jnp.float32)
    y = y + fc2b_ref[...]
    m = jnp.max(y, axis=1, keepdims=True)
    ex = jnp.exp(y - m)
    out_ref[...] = (y - m) - jnp.log(jnp.sum(ex, axis=1, keepdims=True))


@functools.cache
def _make_pool_head(n, g, h, out):
    return pl.pallas_call(
        functools.partial(_pool_head_body, n, g),
        out_shape=jax.ShapeDtypeStruct((g, out), jnp.float32),
    )


def kernel(x, edge_index, batch, c1_W1, c1_b1, c1_g, c1_be, c1_W2, c1_b2,
           c2_W1, c2_b1, c2_g, c2_be, c2_W2, c2_b2, c3_W1, c3_b1, c3_g,
           c3_be, c3_W2, c3_b2, fc1_W, fc1_b, fc2_W, fc2_b):
    n, d = x.shape
    e = edge_index.shape[1]
    h = c1_W1.shape[1]
    g = 64
    out = fc2_W.shape[1]
    _, n_pad = _pad_rows(n)

    src2d = edge_index[0].reshape(e // CH, CH)
    dst2d = edge_index[1].reshape(e // CH, CH)

    pad = ((0, n_pad - n), (0, 0))
    xa = jnp.pad(x[:, :h], pad)
    xb = jnp.pad(x[:, h:], pad)

    agg = _make_agg(n_pad, h, e)
    dense1 = _make_gin_dense1(n, n_pad, h)
    dense23 = _make_gin_dense(n, n_pad, h)
    pool_head = _make_pool_head(n, g, h, out)

    pa = agg(xa, src2d, dst2d)
    pb = agg(xb, src2d, dst2d)
    h1 = dense1(x, pa, pb, c1_W1, c1_b1.reshape(1, h), c1_g.reshape(1, h),
                c1_be.reshape(1, h), c1_W2, c1_b2.reshape(1, h))

    def layer(xin, w1, b1, gm, be, w2, b2):
        parts = agg(xin, src2d, dst2d)
        return dense23(xin, parts, w1, b1.reshape(1, h), gm.reshape(1, h),
                       be.reshape(1, h), w2, b2.reshape(1, h))

    h2 = layer(h1, c2_W1, c2_b1, c2_g, c2_be, c2_W2, c2_b2)
    h3 = layer(h2, c3_W1, c3_b1, c3_g, c3_be, c3_W2, c3_b2)

    return pool_head(h1, h2, h3, batch.reshape(1, n), fc1_W,
                     fc1_b.reshape(1, 3 * h), fc2_W, fc2_b.reshape(1, out))


# speedup vs baseline: 10.5796x; 1.0468x over previous
import functools

import jax
import jax.numpy as jnp
from jax import lax
from jax.experimental import pallas as pl
from jax.experimental.pallas import tpu as pltpu
from jax.experimental.pallas import tpu_sc as plsc

NC = 2
NS = 16
NW = NC * NS
CH = 125
GRP = 4
ZB = 64


def _pad_rows(n):
    rows_per_tile = -(-(-(-n // NS)) // ZB) * ZB
    return rows_per_tile, rows_per_tile * NS


@functools.cache
def _make_agg(n, n_pad, d, e):
    chunks_total = e // CH
    chunks_per_tile = chunks_total // NW
    rows_per_tile = n_pad // NS
    reps = rows_per_tile // ZB
    niter = chunks_per_tile // (2 * GRP)

    mesh = plsc.VectorSubcoreMesh(
        core_axis_name="c", subcore_axis_name="s", num_cores=NC, num_subcores=NS
    )

    @functools.partial(
        pl.kernel,
        out_type=jax.ShapeDtypeStruct((NC, n_pad, d), jnp.float32),
        mesh=mesh,
        scratch_types=[
            pltpu.VMEM((chunks_per_tile, CH), jnp.int32),
            pltpu.VMEM((chunks_per_tile, CH), jnp.int32),
            pltpu.VMEM((2, GRP, CH, d), jnp.float32),
            pltpu.VMEM((ZB, d), jnp.float32),
            pltpu.VMEM_SHARED((n_pad, d), jnp.float32),
            pltpu.SemaphoreType.DMA,
            pltpu.SemaphoreType.DMA,
            pltpu.SemaphoreType.DMA,
            pltpu.SemaphoreType.DMA,
            pltpu.SemaphoreType.DMA,
        ],
        compiler_params=pltpu.CompilerParams(use_tc_tiling_on_sc=False),
    )
    def agg(xb_hbm, src_hbm, dst_hbm, out_hbm, src_v, dst_v, rows_v, zb_v,
            acc_sh, gsem0, gsem1, ssem0, ssem1, isem):
        c = lax.axis_index("c")
        s = lax.axis_index("s")
        wid = c * NS + s
        my_rows = pl.ds(s * rows_per_tile, rows_per_tile)

        pltpu.sync_copy(
            src_hbm.at[pl.ds(wid * chunks_per_tile, chunks_per_tile)], src_v
        )
        pltpu.sync_copy(
            dst_hbm.at[pl.ds(wid * chunks_per_tile, chunks_per_tile)], dst_v
        )

        def zrow(i, carry):
            for jj in range(d // 16):
                zb_v[i, pl.ds(jj * 16, 16)] = jnp.zeros((16,), jnp.float32)
            return carry

        lax.fori_loop(0, ZB, zrow, 0)
        for r in range(reps):
            pltpu.sync_copy(zb_v, acc_sh.at[pl.ds(s * rows_per_tile + r * ZB, ZB)])
        plsc.subcore_barrier()

        def fire_gathers(half, grp, sem):
            return [
                pltpu.async_copy(
                    xb_hbm.at[src_v.at[grp * GRP + b]], rows_v.at[half, b],
                    sem)
                for b in range(GRP)
            ]

        def fire_scatters(half, grp, sem):
            return [
                pltpu.async_copy(
                    rows_v.at[half, b], acc_sh.at[dst_v.at[grp * GRP + b]],
                    sem, add=True)
                for b in range(GRP)
            ]

        def body(i, carry):
            ga = 2 * i
            gb = 2 * i + 1
            gA = fire_gathers(0, ga, gsem0)
            for cp in gA:
                cp.wait()
            sA = fire_scatters(0, ga, ssem0)
            gB = fire_gathers(1, gb, gsem1)
            for cp in gB:
                cp.wait()
            for cp in sA:
                cp.wait()
            sB = fire_scatters(1, gb, ssem1)
            for cp in sB:
                cp.wait()
            return carry

        lax.fori_loop(0, niter, body, 0)
        plsc.subcore_barrier()

        pltpu.sync_copy(acc_sh.at[my_rows], out_hbm.at[c].at[my_rows])

    return agg


def _gin_mlp(n, sfull, w1_ref, b1_ref, g_ref, be_ref, w2_ref, b2_ref,
             out_ref):
    t = jnp.dot(sfull, w1_ref[...], preferred_element_type=jnp.float32)
    t = t + b1_ref[...]
    mu = jnp.sum(t, axis=0, keepdims=True) * (1.0 / n)
    ctr = t - mu
    var = jnp.sum(ctr * ctr, axis=0, keepdims=True) * (1.0 / n)
    h = ctr * lax.rsqrt(var + 1e-5) * g_ref[...] + be_ref[...]
    h = jnp.maximum(h, 0.0)
    h = jnp.dot(h, w2_ref[...], preferred_element_type=jnp.float32) + b2_ref[...]
    out_ref[...] = jnp.maximum(h, 0.0)


def _gin_dense1_body(n, x_ref, pa_ref, pb_ref, w1_ref, b1_ref, g_ref,
                     be_ref, w2_ref, b2_ref, out_ref):
    agg = jnp.concatenate(
        [pa_ref[0][:n] + pa_ref[1][:n], pb_ref[0][:n] + pb_ref[1][:n]], axis=1
    )
    sfull = x_ref[...] + agg
    _gin_mlp(n, sfull, w1_ref, b1_ref, g_ref, be_ref, w2_ref, b2_ref, out_ref)


def _gin_dense_body(n, x_ref, a_ref, w1_ref, b1_ref, g_ref, be_ref,
                    w2_ref, b2_ref, out_ref):
    sfull = x_ref[...] + a_ref[0][:n] + a_ref[1][:n]
    _gin_mlp(n, sfull, w1_ref, b1_ref, g_ref, be_ref, w2_ref, b2_ref, out_ref)


@functools.cache
def _make_gin_dense1(n, h):
    return pl.pallas_call(
        functools.partial(_gin_dense1_body, n),
        out_shape=jax.ShapeDtypeStruct((n, h), jnp.float32),
    )


@functools.cache
def _make_gin_dense(n, h):
    return pl.pallas_call(
        functools.partial(_gin_dense_body, n),
        out_shape=jax.ShapeDtypeStruct((n, h), jnp.float32),
    )


def _pool_head_body(n, g, h1_ref, h2_ref, h3_ref, batch_ref, fc1w_ref,
                    fc1b_ref, fc2w_ref, fc2b_ref, out_ref):
    b = batch_ref[...]
    gids = lax.broadcasted_iota(jnp.int32, (g, n), 0)
    sel = jnp.where(gids == b, 1.0, 0.0)
    p1 = jnp.dot(sel, h1_ref[...], preferred_element_type=jnp.float32)
    p2 = jnp.dot(sel, h2_ref[...], preferred_element_type=jnp.float32)
    p3 = jnp.dot(sel, h3_ref[...], preferred_element_type=jnp.float32)
    cat = jnp.concatenate([p1, p2, p3], axis=1)
    y = jnp.dot(cat, fc1w_ref[...], preferred_element_type=jnp.float32)
    y = jnp.maximum(y + fc1b_ref[...], 0.0)
    y = jnp.dot(y, fc2w_ref[...], preferred_element_type=jnp.float32)
    y = y + fc2b_ref[...]
    m = jnp.max(y, axis=1, keepdims=True)
    ex = jnp.exp(y - m)
    out_ref[...] = (y - m) - jnp.log(jnp.sum(ex, axis=1, keepdims=True))


@functools.cache
def _make_pool_head(n, g, h, out):
    return pl.pallas_call(
        functools.partial(_pool_head_body, n, g),
        out_shape=jax.ShapeDtypeStruct((g, out), jnp.float32),
    )


def kernel(x, edge_index, batch, c1_W1, c1_b1, c1_g, c1_be, c1_W2, c1_b2,
           c2_W1, c2_b1, c2_g, c2_be, c2_W2, c2_b2, c3_W1, c3_b1, c3_g,
           c3_be, c3_W2, c3_b2, fc1_W, fc1_b, fc2_W, fc2_b):
    n, d = x.shape
    e = edge_index.shape[1]
    h = c1_W1.shape[1]
    g = 64
    out = fc2_W.shape[1]
    _, n_pad = _pad_rows(n)

    src2d = edge_index[0].reshape(e // CH, CH)
    dst2d = edge_index[1].reshape(e // CH, CH)

    xa = x[:, :h]
    xb = x[:, h:]

    agg = _make_agg(n, n_pad, h, e)
    dense1 = _make_gin_dense1(n, h)
    dense23 = _make_gin_dense(n, h)
    pool_head = _make_pool_head(n, g, h, out)

    pa = agg(xa, src2d, dst2d)
    pb = agg(xb, src2d, dst2d)
    h1 = dense1(x, pa, pb, c1_W1, c1_b1.reshape(1, h), c1_g.reshape(1, h),
                c1_be.reshape(1, h), c1_W2, c1_b2.reshape(1, h))

    def layer(xin, w1, b1, gm, be, w2, b2):
        parts = agg(xin, src2d, dst2d)
        return dense23(xin, parts, w1, b1.reshape(1, h), gm.reshape(1, h),
                       be.reshape(1, h), w2, b2.reshape(1, h))

    h2 = layer(h1, c2_W1, c2_b1, c2_g, c2_be, c2_W2, c2_b2)
    h3 = layer(h2, c3_W1, c3_b1, c3_g, c3_be, c3_W2, c3_b2)

    return pool_head(h1, h2, h3, batch.reshape(1, n), fc1_W,
                     fc1_b.reshape(1, 3 * h), fc2_W, fc2_b.reshape(1, out))


# speedup vs baseline: 10.6968x; 1.0111x over previous
import functools

import jax
import jax.numpy as jnp
from jax import lax
from jax.experimental import pallas as pl
from jax.experimental.pallas import tpu as pltpu
from jax.experimental.pallas import tpu_sc as plsc

NC = 2
NS = 16
NW = NC * NS
CH = 125
GRP = 4
ZB = 64


def _pad_rows(n):
    rows_per_tile = -(-(-(-n // NS)) // ZB) * ZB
    return rows_per_tile, rows_per_tile * NS


@functools.cache
def _make_agg(n, n_pad, d, e):
    chunks_total = e // CH
    chunks_per_tile = chunks_total // NW
    rows_per_tile = n_pad // NS
    reps = rows_per_tile // ZB
    niter = chunks_per_tile // (2 * GRP)

    mesh = plsc.VectorSubcoreMesh(
        core_axis_name="c", subcore_axis_name="s", num_cores=NC, num_subcores=NS
    )

    @functools.partial(
        pl.kernel,
        out_type=jax.ShapeDtypeStruct((NC, n_pad, d), jnp.float32),
        mesh=mesh,
        scratch_types=[
            pltpu.VMEM((chunks_per_tile, CH), jnp.int32),
            pltpu.VMEM((chunks_per_tile, CH), jnp.int32),
            pltpu.VMEM((2, GRP, CH, d), jnp.float32),
            pltpu.VMEM((ZB, d), jnp.float32),
            pltpu.VMEM_SHARED((n_pad, d), jnp.float32),
            pltpu.SemaphoreType.DMA,
            pltpu.SemaphoreType.DMA,
            pltpu.SemaphoreType.DMA,
            pltpu.SemaphoreType.DMA,
            pltpu.SemaphoreType.DMA,
        ],
        compiler_params=pltpu.CompilerParams(use_tc_tiling_on_sc=False),
    )
    def agg(xb_hbm, src_hbm, dst_hbm, out_hbm, src_v, dst_v, rows_v, zb_v,
            acc_sh, gsem0, gsem1, ssem0, ssem1, isem):
        c = lax.axis_index("c")
        s = lax.axis_index("s")
        wid = c * NS + s
        my_rows = pl.ds(s * rows_per_tile, rows_per_tile)

        pltpu.sync_copy(
            src_hbm.at[pl.ds(wid * chunks_per_tile, chunks_per_tile)], src_v
        )
        pltpu.sync_copy(
            dst_hbm.at[pl.ds(wid * chunks_per_tile, chunks_per_tile)], dst_v
        )

        def zrow(i, carry):
            for jj in range(d // 16):
                zb_v[i, pl.ds(jj * 16, 16)] = jnp.zeros((16,), jnp.float32)
            return carry

        lax.fori_loop(0, ZB, zrow, 0)
        for r in range(reps):
            pltpu.sync_copy(zb_v, acc_sh.at[pl.ds(s * rows_per_tile + r * ZB, ZB)])
        plsc.subcore_barrier()

        def fire_gathers(half, grp, sem):
            return [
                pltpu.async_copy(
                    xb_hbm.at[src_v.at[grp * GRP + b]], rows_v.at[half, b],
                    sem)
                for b in range(GRP)
            ]

        def fire_scatters(half, grp, sem):
            return [
                pltpu.async_copy(
                    rows_v.at[half, b], acc_sh.at[dst_v.at[grp * GRP + b]],
                    sem, add=True)
                for b in range(GRP)
            ]

        def body(i, carry):
            ga = 2 * i
            gb = 2 * i + 1
            gA = fire_gathers(0, ga, gsem0)
            for cp in gA:
                cp.wait()
            sA = fire_scatters(0, ga, ssem0)
            gB = fire_gathers(1, gb, gsem1)
            for cp in gB:
                cp.wait()
            for cp in sA:
                cp.wait()
            sB = fire_scatters(1, gb, ssem1)
            for cp in sB:
                cp.wait()
            return carry

        lax.fori_loop(0, niter, body, 0)
        plsc.subcore_barrier()

        pltpu.sync_copy(acc_sh.at[my_rows], out_hbm.at[c].at[my_rows])

    return agg


def _gin_mlp(n, sfull, w1_ref, b1_ref, g_ref, be_ref, w2_ref, b2_ref,
             out_ref):
    t = jnp.dot(sfull, w1_ref[...], preferred_element_type=jnp.float32)
    t = t + b1_ref[...]
    mu = jnp.sum(t, axis=0, keepdims=True) * (1.0 / n)
    ctr = t - mu
    var = jnp.sum(ctr * ctr, axis=0, keepdims=True) * (1.0 / n)
    h = ctr * lax.rsqrt(var + 1e-5) * g_ref[...] + be_ref[...]
    h = jnp.maximum(h, 0.0)
    h = jnp.dot(h, w2_ref[...], preferred_element_type=jnp.float32) + b2_ref[...]
    out_ref[...] = jnp.maximum(h, 0.0)


def _gin_dense1_body(n, x_ref, pa_ref, pb_ref, w1_ref, b1_ref, g_ref,
                     be_ref, w2_ref, b2_ref, out_ref):
    agg = jnp.concatenate(
        [pa_ref[0][:n] + pa_ref[1][:n], pb_ref[0][:n] + pb_ref[1][:n]], axis=1
    )
    sfull = x_ref[...] + agg
    _gin_mlp(n, sfull, w1_ref, b1_ref, g_ref, be_ref, w2_ref, b2_ref, out_ref)


def _gin_dense_body(n, x_ref, a_ref, w1_ref, b1_ref, g_ref, be_ref,
                    w2_ref, b2_ref, out_ref):
    sfull = x_ref[...] + a_ref[0][:n] + a_ref[1][:n]
    _gin_mlp(n, sfull, w1_ref, b1_ref, g_ref, be_ref, w2_ref, b2_ref, out_ref)


@functools.cache
def _make_gin_dense1(n, h):
    return pl.pallas_call(
        functools.partial(_gin_dense1_body, n),
        out_shape=jax.ShapeDtypeStruct((n, h), jnp.float32),
    )


@functools.cache
def _make_gin_dense(n, h):
    return pl.pallas_call(
        functools.partial(_gin_dense_body, n),
        out_shape=jax.ShapeDtypeStruct((n, h), jnp.float32),
    )


def _dense3_pool_head_body(n, g, x_ref, a_ref, w1_ref, b1_ref, g_ref, be_ref,
                           w2_ref, b2_ref, h1_ref, h2_ref, batch_ref,
                           fc1w_ref, fc1b_ref, fc2w_ref, fc2b_ref, out_ref):
    sfull = x_ref[...] + a_ref[0][:n] + a_ref[1][:n]
    t = jnp.dot(sfull, w1_ref[...], preferred_element_type=jnp.float32)
    t = t + b1_ref[...]
    mu = jnp.sum(t, axis=0, keepdims=True) * (1.0 / n)
    ctr = t - mu
    var = jnp.sum(ctr * ctr, axis=0, keepdims=True) * (1.0 / n)
    hh = ctr * lax.rsqrt(var + 1e-5) * g_ref[...] + be_ref[...]
    hh = jnp.maximum(hh, 0.0)
    hh = jnp.dot(hh, w2_ref[...], preferred_element_type=jnp.float32)
    h3 = jnp.maximum(hh + b2_ref[...], 0.0)

    b = batch_ref[...]
    gids = lax.broadcasted_iota(jnp.int32, (g, n), 0)
    sel = jnp.where(gids == b, 1.0, 0.0)
    p1 = jnp.dot(sel, h1_ref[...], preferred_element_type=jnp.float32)
    p2 = jnp.dot(sel, h2_ref[...], preferred_element_type=jnp.float32)
    p3 = jnp.dot(sel, h3, preferred_element_type=jnp.float32)
    cat = jnp.concatenate([p1, p2, p3], axis=1)
    y = jnp.dot(cat, fc1w_ref[...], preferred_element_type=jnp.float32)
    y = jnp.maximum(y + fc1b_ref[...], 0.0)
    y = jnp.dot(y, fc2w_ref[...], preferred_element_type=jnp.float32)
    y = y + fc2b_ref[...]
    m = jnp.max(y, axis=1, keepdims=True)
    ex = jnp.exp(y - m)
    out_ref[...] = (y - m) - jnp.log(jnp.sum(ex, axis=1, keepdims=True))


@functools.cache
def _make_dense3_pool_head(n, g, out):
    return pl.pallas_call(
        functools.partial(_dense3_pool_head_body, n, g),
        out_shape=jax.ShapeDtypeStruct((g, out), jnp.float32),
    )


def kernel(x, edge_index, batch, c1_W1, c1_b1, c1_g, c1_be, c1_W2, c1_b2,
           c2_W1, c2_b1, c2_g, c2_be, c2_W2, c2_b2, c3_W1, c3_b1, c3_g,
           c3_be, c3_W2, c3_b2, fc1_W, fc1_b, fc2_W, fc2_b):
    n, d = x.shape
    e = edge_index.shape[1]
    h = c1_W1.shape[1]
    g = 64
    out = fc2_W.shape[1]
    _, n_pad = _pad_rows(n)

    src2d = edge_index[0].reshape(e // CH, CH)
    dst2d = edge_index[1].reshape(e // CH, CH)

    xa = x[:, :h]
    xb = x[:, h:]

    agg = _make_agg(n, n_pad, h, e)
    dense1 = _make_gin_dense1(n, h)
    dense23 = _make_gin_dense(n, h)
    dense3_pool = _make_dense3_pool_head(n, g, out)

    pa = agg(xa, src2d, dst2d)
    pb = agg(xb, src2d, dst2d)
    h1 = dense1(x, pa, pb, c1_W1, c1_b1.reshape(1, h), c1_g.reshape(1, h),
                c1_be.reshape(1, h), c1_W2, c1_b2.reshape(1, h))

    parts2 = agg(h1, src2d, dst2d)
    h2 = dense23(h1, parts2, c2_W1, c2_b1.reshape(1, h), c2_g.reshape(1, h),
                 c2_be.reshape(1, h), c2_W2, c2_b2.reshape(1, h))

    parts3 = agg(h2, src2d, dst2d)
    return dense3_pool(h2, parts3, c3_W1, c3_b1.reshape(1, h),
                       c3_g.reshape(1, h), c3_be.reshape(1, h), c3_W2,
                       c3_b2.reshape(1, h), h1, h2, batch.reshape(1, n),
                       fc1_W, fc1_b.reshape(1, 3 * h), fc2_W,
                       fc2_b.reshape(1, out))
